# Initial kernel scaffold; baseline (speedup 1.0000x reference)
import jax
import jax.numpy as jnp
from jax.experimental import pallas as pl


def kernel(input_ids, weight):
    raise NotImplementedError("write your pallas kernel here")


# speedup vs baseline: 2.9653x; 2.9653x over previous
import functools

import jax
import jax.numpy as jnp
from jax import lax
from jax.experimental import pallas as pl
from jax.experimental.pallas import tpu as pltpu
from jax.experimental.pallas import tpu_sc as plsc

HIDDEN = 128
NUM_WORKERS = 32
CHUNK = 128


def kernel(input_ids, weight):
    B, S = input_ids.shape
    total = B * S
    bpw = total // NUM_WORKERS
    nch = bpw // CHUNK
    idx = input_ids.reshape(NUM_WORKERS, nch, CHUNK).astype(jnp.int32)

    mesh = plsc.VectorSubcoreMesh(core_axis_name="c", subcore_axis_name="s")

    @functools.partial(
        pl.kernel,
        mesh=mesh,
        out_type=jax.ShapeDtypeStruct((total, HIDDEN), jnp.float32),
        scratch_types=[
            pltpu.VMEM((nch, CHUNK), jnp.int32),
            pltpu.VMEM((CHUNK, HIDDEN), jnp.float32),
            pltpu.SemaphoreType.DMA,
        ],
    )
    def emb(table_hbm, idx_hbm, out_hbm, idx_v, rows_v, sem):
        wid = lax.axis_index("s") * 2 + lax.axis_index("c")
        base = wid * bpw
        pltpu.sync_copy(idx_hbm.at[wid], idx_v)

        def body(j, carry):
            pltpu.async_copy(table_hbm.at[idx_v.at[j]], rows_v, sem).wait()
            pltpu.sync_copy(rows_v, out_hbm.at[pl.ds(base + j * CHUNK, CHUNK)])
            return carry

        lax.fori_loop(0, nch, body, 0)

    out = emb(weight, idx)
    return out.reshape(B, S, HIDDEN)


# speedup vs baseline: 3.3443x; 1.1278x over previous
import functools

import jax
import jax.numpy as jnp
from jax import lax
from jax.experimental import pallas as pl
from jax.experimental.pallas import tpu as pltpu
from jax.experimental.pallas import tpu_sc as plsc

HIDDEN = 128
NUM_WORKERS = 32
CHUNK = 128


def kernel(input_ids, weight):
    B, S = input_ids.shape
    total = B * S
    bpw = total // NUM_WORKERS
    nch = bpw // CHUNK
    idx = input_ids.reshape(NUM_WORKERS, nch, CHUNK).astype(jnp.int32)

    mesh = plsc.VectorSubcoreMesh(core_axis_name="c", subcore_axis_name="s")

    @functools.partial(
        pl.kernel,
        mesh=mesh,
        out_type=jax.ShapeDtypeStruct((total, HIDDEN), jnp.float32),
        scratch_types=[
            pltpu.VMEM((nch, CHUNK), jnp.int32),
            pltpu.VMEM((2, CHUNK, HIDDEN), jnp.float32),
            pltpu.SemaphoreType.DMA,
            pltpu.SemaphoreType.DMA,
            pltpu.SemaphoreType.DMA,
            pltpu.SemaphoreType.DMA,
        ],
    )
    def emb(table_hbm, idx_hbm, out_hbm, idx_v, rows_v, in0, in1, out0, out1):
        wid = lax.axis_index("s") * 2 + lax.axis_index("c")
        base = wid * bpw
        sem_in = (in0, in1)
        sem_out = (out0, out1)

        pltpu.sync_copy(idx_hbm.at[wid], idx_v)

        def start_gather(g, b):
            pltpu.make_async_copy(
                table_hbm.at[idx_v.at[g]], rows_v.at[b], sem_in[b]).start()

        def wait_gather(b):
            pltpu.make_async_copy(
                table_hbm.at[idx_v.at[0]], rows_v.at[b], sem_in[b]).wait()

        def start_out(g, b):
            pltpu.make_async_copy(
                rows_v.at[b], out_hbm.at[pl.ds(base + g * CHUNK, CHUNK)],
                sem_out[b]).start()

        def wait_out(b):
            pltpu.make_async_copy(
                rows_v.at[b], out_hbm.at[pl.ds(base, CHUNK)],
                sem_out[b]).wait()

        start_gather(0, 0)
        start_gather(1, 1)
        wait_gather(0)
        start_out(0, 0)

        def body(k, carry):
            for off in (1, 2):
                g = 2 * k + off
                b = off % 2
                nb = 1 - b
                wait_out(nb)
                start_gather(g + 1, nb)
                wait_gather(b)
                start_out(g, b)
            return carry

        lax.fori_loop(0, (nch - 2) // 2, body, 0)

        wait_gather(1)
        start_out(nch - 1, 1)
        wait_out(0)
        wait_out(1)

    out = emb(weight, idx)
    return out.reshape(B, S, HIDDEN)


# speedup vs baseline: 5.9084x; 1.7667x over previous
import functools

import jax
import jax.numpy as jnp
from jax import lax
from jax.experimental import pallas as pl
from jax.experimental.pallas import tpu as pltpu
from jax.experimental.pallas import tpu_sc as plsc

HIDDEN = 128
NUM_WORKERS = 32
GROUP = 4


def kernel(input_ids, weight):
    B, S = input_ids.shape
    bpw = B // NUM_WORKERS
    nsteps = bpw // GROUP
    idx = input_ids.astype(jnp.int32)

    mesh = plsc.VectorSubcoreMesh(core_axis_name="c", subcore_axis_name="s")

    @functools.partial(
        pl.kernel,
        mesh=mesh,
        out_type=jax.ShapeDtypeStruct((B, S, HIDDEN), jnp.float32),
        compiler_params=pltpu.CompilerParams(use_tc_tiling_on_sc=True),
        scratch_types=[
            pltpu.VMEM((bpw, S), jnp.int32),
            pltpu.VMEM((2, GROUP, S, HIDDEN), jnp.float32),
            pltpu.SemaphoreType.DMA,
            pltpu.SemaphoreType.DMA,
            pltpu.SemaphoreType.DMA,
            pltpu.SemaphoreType.DMA,
        ],
    )
    def emb(table_hbm, idx_hbm, out_hbm, idx_v, rows_v, in0, in1, out0, out1):
        wid = lax.axis_index("s") * 2 + lax.axis_index("c")
        base = wid * bpw
        sem_in = (in0, in1)
        sem_out = (out0, out1)

        pltpu.sync_copy(idx_hbm.at[pl.ds(base, bpw)], idx_v)

        def start_gathers(j, b):
            for i in range(GROUP):
                pltpu.make_async_copy(
                    table_hbm.at[idx_v.at[j * GROUP + i]],
                    rows_v.at[b, i], sem_in[b]).start()

        def wait_gathers(b):
            for i in range(GROUP):
                pltpu.make_async_copy(
                    table_hbm.at[idx_v.at[0]],
                    rows_v.at[b, i], sem_in[b]).wait()

        def start_write(j, b):
            pltpu.make_async_copy(
                rows_v.at[b], out_hbm.at[pl.ds(base + j * GROUP, GROUP)],
                sem_out[b]).start()

        def wait_write(b):
            pltpu.make_async_copy(
                rows_v.at[b], out_hbm.at[pl.ds(base, GROUP)],
                sem_out[b]).wait()

        start_gathers(0, 0)
        start_gathers(1, 1)
        wait_gathers(0)
        start_write(0, 0)

        def body(k, carry):
            for off in (1, 2):
                j = 2 * k + off
                b = off % 2
                nb = 1 - b
                wait_write(nb)
                start_gathers(j + 1, nb)
                wait_gathers(b)
                start_write(j, b)
            return carry

        lax.fori_loop(0, (nsteps - 2) // 2, body, 0)

        wait_gathers(1)
        start_write(nsteps - 1, 1)
        wait_write(0)
        wait_write(1)

    out = emb(weight, idx)
    return out


# speedup vs baseline: 10.4175x; 1.7632x over previous
import functools

import jax
import jax.numpy as jnp
from jax import lax
from jax.experimental import pallas as pl
from jax.experimental.pallas import tpu as pltpu
from jax.experimental.pallas import tpu_sc as plsc

HIDDEN = 128
NUM_WORKERS = 32


def kernel(input_ids, weight):
    B, S = input_ids.shape
    bpw = B // NUM_WORKERS
    idx_t = input_ids.astype(jnp.int32).T

    mesh = plsc.VectorSubcoreMesh(core_axis_name="c", subcore_axis_name="s")

    @functools.partial(
        pl.kernel,
        mesh=mesh,
        out_type=jax.ShapeDtypeStruct((S, B, HIDDEN), jnp.float32),
        compiler_params=pltpu.CompilerParams(use_tc_tiling_on_sc=True),
        scratch_types=[
            pltpu.VMEM((S, bpw), jnp.int32),
            pltpu.VMEM((2, bpw, HIDDEN), jnp.float32),
            pltpu.SemaphoreType.DMA,
            pltpu.SemaphoreType.DMA,
            pltpu.SemaphoreType.DMA,
            pltpu.SemaphoreType.DMA,
        ],
    )
    def emb(table_hbm, idx_hbm, out_hbm, idx_v, rows_v, in0, in1, out0, out1):
        wid = lax.axis_index("s") * 2 + lax.axis_index("c")
        base = wid * bpw
        sem_in = (in0, in1)
        sem_out = (out0, out1)

        pltpu.sync_copy(idx_hbm.at[:, pl.ds(base, bpw)], idx_v)

        def start_gather(s, b):
            pltpu.make_async_copy(
                table_hbm.at[idx_v.at[s]], rows_v.at[b], sem_in[b]).start()

        def wait_gather(b):
            pltpu.make_async_copy(
                table_hbm.at[idx_v.at[0]], rows_v.at[b], sem_in[b]).wait()

        def start_write(s, b):
            pltpu.make_async_copy(
                rows_v.at[b], out_hbm.at[s, pl.ds(base, bpw)],
                sem_out[b]).start()

        def wait_write(b):
            pltpu.make_async_copy(
                rows_v.at[b], out_hbm.at[0, pl.ds(base, bpw)],
                sem_out[b]).wait()

        start_gather(0, 0)
        start_gather(1, 1)
        wait_gather(0)
        start_write(0, 0)

        def body(k, carry):
            for off in (1, 2):
                s = 2 * k + off
                b = off % 2
                nb = 1 - b
                wait_write(nb)
                start_gather(s + 1, nb)
                wait_gather(b)
                start_write(s, b)
            return carry

        lax.fori_loop(0, (S - 2) // 2, body, 0)

        wait_gather(1)
        start_write(S - 1, 1)
        wait_write(0)
        wait_write(1)

    out = emb(weight, idx_t)
    return out.transpose(1, 0, 2)


# speedup vs baseline: 10.6631x; 1.0236x over previous
import functools

import jax
import jax.numpy as jnp
from jax import lax
from jax.experimental import pallas as pl
from jax.experimental.pallas import tpu as pltpu
from jax.experimental.pallas import tpu_sc as plsc

HIDDEN = 128
NUM_WORKERS = 32
NBUF = 4


def kernel(input_ids, weight):
    B, S = input_ids.shape
    bpw = B // NUM_WORKERS
    idx_t = input_ids.astype(jnp.int32).T

    mesh = plsc.VectorSubcoreMesh(core_axis_name="c", subcore_axis_name="s")

    @functools.partial(
        pl.kernel,
        mesh=mesh,
        out_type=jax.ShapeDtypeStruct((S, B, HIDDEN), jnp.float32),
        compiler_params=pltpu.CompilerParams(use_tc_tiling_on_sc=True),
        scratch_types=[
            pltpu.VMEM((S, bpw), jnp.int32),
            pltpu.VMEM((NBUF, bpw, HIDDEN), jnp.float32),
            [pltpu.SemaphoreType.DMA] * NBUF,
            [pltpu.SemaphoreType.DMA] * NBUF,
        ],
    )
    def emb(table_hbm, idx_hbm, out_hbm, idx_v, rows_v, sem_in, sem_out):
        wid = lax.axis_index("s") * 2 + lax.axis_index("c")
        base = wid * bpw

        pltpu.sync_copy(idx_hbm.at[:, pl.ds(base, bpw)], idx_v)

        def start_gather(s, b):
            pltpu.make_async_copy(
                table_hbm.at[idx_v.at[s]], rows_v.at[b], sem_in[b]).start()

        def wait_gather(b):
            pltpu.make_async_copy(
                table_hbm.at[idx_v.at[0]], rows_v.at[b], sem_in[b]).wait()

        def start_write(s, b):
            pltpu.make_async_copy(
                rows_v.at[b], out_hbm.at[s, pl.ds(base, bpw)],
                sem_out[b]).start()

        def wait_write(b):
            pltpu.make_async_copy(
                rows_v.at[b], out_hbm.at[0, pl.ds(base, bpw)],
                sem_out[b]).wait()

        start_gather(0, 0)
        start_gather(1, 1)
        start_gather(2, 2)
        wait_gather(0)
        start_write(0, 0)
        start_gather(3, 3)
        wait_gather(1)
        start_write(1, 1)

        def step(s, b):
            nb = (b + 2) % NBUF
            wait_write(nb)
            start_gather(s + 2, nb)
            wait_gather(b)
            start_write(s, b)

        def body(k, carry):
            for off in range(NBUF):
                s = NBUF * k + 2 + off
                step(s, (2 + off) % NBUF)
            return carry

        lax.fori_loop(0, (S - 6) // NBUF, body, 0)

        step(S - 4, (S - 4) % NBUF)
        step(S - 3, (S - 3) % NBUF)

        wait_gather((S - 2) % NBUF)
        start_write(S - 2, (S - 2) % NBUF)
        wait_gather((S - 1) % NBUF)
        start_write(S - 1, (S - 1) % NBUF)
        for b in range(NBUF):
            wait_write(b)

    out = emb(weight, idx_t)
    return out.transpose(1, 0, 2)
